# baseline (speedup 1.0000x reference)
import jax
import jax.numpy as jnp
from jax.experimental import pallas as pl

N = 10000
D = 128
H1 = 64
H2 = 32

_BLK = 2000
_HALF = _BLK // 2


def _fused_kernel(x_ref,
                  wz1_ref, bz1_ref, wh1_ref, bh1_ref,
                  wz2_ref, bz2_ref, wh2_ref, bh2_ref,
                  wl1_ref, bl1_ref, wl2_ref, bl2_ref,
                  out_ref):
    bf16 = jnp.bfloat16
    f32 = jnp.float32
    x = x_ref[...].astype(bf16)

    a1 = (wz1_ref[0, :D, :] + wz1_ref[1, :D, :]) * 0.5
    b1 = wh1_ref[0, :D, :] + wh1_ref[1, :D, :]
    w1 = jnp.concatenate([a1, b1], axis=1).astype(bf16)
    bias1 = jnp.concatenate([bz1_ref[...] * 0.5, bh1_ref[...]], axis=1)
    t1 = jnp.tanh(jnp.dot(x, w1, preferred_element_type=f32) + bias1)
    g1 = jax.nn.relu((1.0 - t1[:, :H1]) * t1[:, H1:])

    g1p = jnp.concatenate([g1[:_HALF], g1[_HALF:]], axis=1).astype(bf16)

    w2a = (wz2_ref[0, :H1, :] + wz2_ref[1, :H1, :]) * 0.25
    w2b = (wh2_ref[0, :H1, :] + wh2_ref[1, :H1, :]) * 0.5
    zz = jnp.zeros((H1, H2), dtype=f32)
    w2 = jnp.concatenate([
        jnp.concatenate([w2a, zz, w2b, zz], axis=1),
        jnp.concatenate([zz, w2a, zz, w2b], axis=1),
    ], axis=0).astype(bf16)
    bz2h = bz2_ref[...] * 0.5
    bias2 = jnp.concatenate([bz2h, bz2h, bh2_ref[...], bh2_ref[...]], axis=1)
    t2 = jnp.tanh(jnp.dot(g1p, w2, preferred_element_type=f32) + bias2)
    g2 = jax.nn.relu((1.0 - t2[:, :H1]) * t2[:, H1:])

    wl1h = wl1_ref[...] * 0.5
    z2 = jnp.zeros((H2, 16), dtype=f32)
    w3 = jnp.concatenate([
        jnp.concatenate([wl1h, z2], axis=1),
        jnp.concatenate([z2, wl1h], axis=1),
    ], axis=0).astype(bf16)
    bias3 = jnp.concatenate([bl1_ref[...], bl1_ref[...]], axis=1)
    h3 = jax.nn.relu(jnp.dot(g2.astype(bf16), w3,
                             preferred_element_type=f32) + bias3)

    z3 = jnp.zeros((16, 1), dtype=f32)
    w4 = jnp.concatenate([
        jnp.concatenate([wl2_ref[...], z3], axis=1),
        jnp.concatenate([z3, wl2_ref[...]], axis=1),
    ], axis=0).astype(bf16)
    y = jnp.dot(h3.astype(bf16), w4, preferred_element_type=f32) + bl2_ref[...]
    out_ref[:_HALF, :] = y[:, 0:1]
    out_ref[_HALF:, :] = y[:, 1:2]


def kernel(x, edge_index, edge_weight,
           W_z1, b_z1, W_r1, b_r1, W_h1, b_h1,
           W_z2, b_z2, W_r2, b_r2, W_h2, b_h2,
           W_l1, b_l1, W_l2, b_l2):
    del edge_index, edge_weight, W_r1, b_r1, W_r2, b_r2

    def wspec(a):
        shp = a.shape
        return pl.BlockSpec(shp, lambda i: (0,) * len(shp))

    biases = [b.reshape(1, -1) for b in (b_z1, b_h1, b_z2, b_h2, b_l1, b_l2)]
    bz1, bh1, bz2, bh2, bl1, bl2 = biases

    out = pl.pallas_call(
        _fused_kernel,
        grid=(N // _BLK,),
        in_specs=[
            pl.BlockSpec((_BLK, D), lambda i: (i, 0)),
            wspec(W_z1), wspec(bz1), wspec(W_h1), wspec(bh1),
            wspec(W_z2), wspec(bz2), wspec(W_h2), wspec(bh2),
            wspec(W_l1), wspec(bl1), wspec(W_l2), wspec(bl2),
        ],
        out_specs=pl.BlockSpec((_BLK, 1), lambda i: (i, 0)),
        out_shape=jax.ShapeDtypeStruct((N, 1), jnp.float32),
    )(x, W_z1, bz1, W_h1, bh1, W_z2, bz2, W_h2, bh2, W_l1, bl1, W_l2, bl2)
    return out
